# Initial kernel scaffold; baseline (speedup 1.0000x reference)
import jax
import jax.numpy as jnp
from jax.experimental import pallas as pl


def kernel(x, p):
    raise NotImplementedError("write your pallas kernel here")


# speedup vs baseline: 537.9991x; 537.9991x over previous
import functools

import jax
import jax.numpy as jnp
import numpy as np
from jax.experimental import pallas as pl
from jax.experimental.pallas import tpu as pltpu

_N_BINS = 32
_R = 1.2
_BOUND = 50.0
_N_LENGTH = 16


def _np_mesh_constants():
    m = _N_BINS / 2
    x1L = _BOUND * (_R - 1.0) / (_R**m - 1.0)
    index = np.arange(0, _N_BINS + 1, dtype=np.float64).reshape(-1, 1) - m
    xr = np.where(index >= 0,
                  (1.0 - _R**index) / (1.0 - _R),
                  (1.0 - _R**np.abs(index)) / (1.0 - _R))
    xr = np.where(index >= 0, x1L * xr, -x1L * xr)
    xr = (xr + _BOUND) / 2.0 / _BOUND
    x1L_s = x1L / 2.0 / _BOUND
    mesh = np.concatenate([np.zeros((1, 1)), xr[1:-1, 0:1], np.ones((1, 1))], 0)
    elmt = (mesh[1:] - mesh[:-1]).reshape(-1, 1)
    return (mesh.astype(np.float32), elmt.astype(np.float32),
            np.float32(x1L_s))


_MESH, _ELMT, _X1L = _np_mesh_constants()
_TRI = (np.arange(_N_BINS)[:, None] > np.arange(_N_BINS)[None, :]).astype(np.float32)
_ACOEF = float((_R - 1.0) / _X1L)
_INV_LOG_R = float(1.0 / np.log(_R))
_D128 = np.tile(_MESH[:_N_BINS, :], (1, 128)).astype(np.float32)


def _prep_kernel(p_ref, elmt_ref, tri_ref, t_ref):
    p = p_ref[...]
    ep = jnp.exp(p)
    elmt = elmt_ref[...]
    w = (elmt[:-1] + elmt[1:]) / 2.0
    s = jnp.sum(ep * w, axis=0, keepdims=True)
    px = ((1.0 - float(_ELMT[0, 0])) / s) * ep
    one = jnp.ones((1, _N_LENGTH), jnp.float32)
    pdf = jnp.concatenate([one, px, one], axis=0)
    cell = (pdf[:-1] + pdf[1:]) / 2.0 * elmt
    f_ref = jnp.dot(tri_ref[...], cell,
                    preferred_element_type=jnp.float32,
                    precision=jax.lax.Precision.HIGHEST)
    b = pdf[:_N_BINS]
    c = (pdf[1:] - pdf[:-1]) / (2.0 * elmt)
    t_ref[...] = jnp.concatenate([f_ref, b, c], axis=0)


def _main_kernel(t_ref, x_ref, o_ref):
    t = t_ref[...]
    d128 = t[96:128]
    x = x_ref[...]
    xs = (x + _BOUND) * (0.5 / _BOUND)
    d = xs - 0.5
    tt = jnp.abs(d) * _ACOEF + 1.0
    mvar = jnp.floor(jnp.log(tt) * _INV_LOG_R)
    kf = jnp.where(d >= 0, 16.0 + mvar, 15.0 - mvar)
    inr = (kf >= 0.0) & (kf <= 31.0)
    ki = jnp.clip(kf, 0.0, 31.0).astype(jnp.int32)
    loc = jnp.bitwise_and(ki, 7)
    cid = jnp.right_shift(ki, 3)

    def gather8(tbl, idx):
        return jnp.take_along_axis(tbl, idx, axis=0, mode="promise_in_bounds")

    a = b = c = dm = None
    for ch in range(4):
        ga = gather8(t[ch * 8:(ch + 1) * 8], loc)
        gb = gather8(t[32 + ch * 8:32 + (ch + 1) * 8], loc)
        gc = gather8(t[64 + ch * 8:64 + (ch + 1) * 8], loc)
        gd = gather8(d128[ch * 8:(ch + 1) * 8], loc)
        if ch == 0:
            a, b, c, dm = ga, gb, gc, gd
        else:
            m = cid == ch
            a = jnp.where(m, ga, a)
            b = jnp.where(m, gb, b)
            c = jnp.where(m, gc, c)
            dm = jnp.where(m, gd, dm)
    xm = xs - dm
    y = a + xm * (b + xm * c)
    res = jnp.where(inr, y, xs)
    o_ref[...] = res * (2.0 * _BOUND) - _BOUND


_BR = 1024


@functools.partial(jax.jit, static_argnames=("interpret",))
def kernel(x, p, interpret=False):
    batch, n_len = x.shape
    total = batch * n_len
    rows = total // 128
    x_r = x.reshape(rows, 128)

    t16 = pl.pallas_call(
        _prep_kernel,
        out_shape=jax.ShapeDtypeStruct((96, _N_LENGTH), jnp.float32),
        interpret=interpret,
    )(p, jnp.asarray(_ELMT), jnp.asarray(_TRI))
    t128 = jnp.concatenate([jnp.tile(t16, (1, 8)), jnp.asarray(_D128)], axis=0)

    grid = rows // _BR
    y_r = pl.pallas_call(
        _main_kernel,
        grid=(grid,),
        in_specs=[
            pl.BlockSpec((128, 128), lambda i: (0, 0)),
            pl.BlockSpec((_BR, 128), lambda i: (i, 0)),
        ],
        out_specs=pl.BlockSpec((_BR, 128), lambda i: (i, 0)),
        out_shape=jax.ShapeDtypeStruct((rows, 128), jnp.float32),
        compiler_params=pltpu.CompilerParams(
            dimension_semantics=("parallel",),
        ),
        interpret=interpret,
    )(t128, x_r)
    return y_r.reshape(batch, n_len)


# speedup vs baseline: 549.2239x; 1.0209x over previous
import functools

import jax
import jax.numpy as jnp
import numpy as np
from jax.experimental import pallas as pl
from jax.experimental.pallas import tpu as pltpu

_N_BINS = 32
_R = 1.2
_BOUND = 50.0
_N_LENGTH = 16


def _np_mesh_constants():
    m = _N_BINS / 2
    x1L = _BOUND * (_R - 1.0) / (_R**m - 1.0)
    index = np.arange(0, _N_BINS + 1, dtype=np.float64).reshape(-1, 1) - m
    xr = np.where(index >= 0,
                  (1.0 - _R**index) / (1.0 - _R),
                  (1.0 - _R**np.abs(index)) / (1.0 - _R))
    xr = np.where(index >= 0, x1L * xr, -x1L * xr)
    xr = (xr + _BOUND) / 2.0 / _BOUND
    x1L_s = x1L / 2.0 / _BOUND
    mesh = np.concatenate([np.zeros((1, 1)), xr[1:-1, 0:1], np.ones((1, 1))], 0)
    elmt = (mesh[1:] - mesh[:-1]).reshape(-1, 1)
    return (mesh.astype(np.float32), elmt.astype(np.float32),
            np.float32(x1L_s))


_MESH, _ELMT, _X1L = _np_mesh_constants()
_TRI = (np.arange(_N_BINS)[:, None] > np.arange(_N_BINS)[None, :]).astype(np.float32)
_ACOEF = float((_R - 1.0) / _X1L)
_INV_LOG_R = float(1.0 / np.log(_R))
_D128 = np.tile(_MESH[:_N_BINS, :], (1, 128)).astype(np.float32)


def _prep_kernel(p_ref, elmt_ref, tri_ref, t_ref):
    p = p_ref[...]
    ep = jnp.exp(p)
    elmt = elmt_ref[...]
    w = (elmt[:-1] + elmt[1:]) / 2.0
    s = jnp.sum(ep * w, axis=0, keepdims=True)
    px = ((1.0 - float(_ELMT[0, 0])) / s) * ep
    one = jnp.ones((1, _N_LENGTH), jnp.float32)
    pdf = jnp.concatenate([one, px, one], axis=0)
    cell = (pdf[:-1] + pdf[1:]) / 2.0 * elmt
    f_ref = jnp.dot(tri_ref[...], cell,
                    preferred_element_type=jnp.float32,
                    precision=jax.lax.Precision.HIGHEST)
    b = pdf[:_N_BINS]
    c = (pdf[1:] - pdf[:-1]) / (2.0 * elmt)
    t_ref[...] = jnp.concatenate([f_ref, b, c], axis=0)


def _compute_block(t, d128, x):
    xs = (x + _BOUND) * (0.5 / _BOUND)
    d = xs - 0.5
    tt = jnp.abs(d) * _ACOEF + 1.0
    mvar = jnp.floor(jnp.log(tt) * _INV_LOG_R)
    kf = jnp.where(d >= 0, 16.0 + mvar, 15.0 - mvar)
    inr = (kf >= 0.0) & (kf <= 31.0)
    ki = jnp.clip(kf, 0.0, 31.0).astype(jnp.int32)
    loc = jnp.bitwise_and(ki, 7)
    cid = jnp.right_shift(ki, 3)

    def gather8(tbl, idx):
        return jnp.take_along_axis(tbl, idx, axis=0, mode="promise_in_bounds")

    a = b = c = dm = None
    for ch in range(4):
        ga = gather8(t[ch * 8:(ch + 1) * 8], loc)
        gb = gather8(t[32 + ch * 8:32 + (ch + 1) * 8], loc)
        gc = gather8(t[64 + ch * 8:64 + (ch + 1) * 8], loc)
        gd = gather8(d128[ch * 8:(ch + 1) * 8], loc)
        if ch == 0:
            a, b, c, dm = ga, gb, gc, gd
        else:
            m = cid == ch
            a = jnp.where(m, ga, a)
            b = jnp.where(m, gb, b)
            c = jnp.where(m, gc, c)
            dm = jnp.where(m, gd, dm)
    xm = xs - dm
    y = a + xm * (b + xm * c)
    res = jnp.where(inr, y, xs)
    return res * (2.0 * _BOUND) - _BOUND


def _main_kernel(t_ref, x_ref, o_ref):
    t = t_ref[...]
    d128 = t[96:128]
    x = jnp.concatenate([x_ref[m::8, :] for m in range(8)], axis=1)
    res = _compute_block(t, d128, x)
    for m in range(8):
        o_ref[m::8, :] = res[:, 16 * m:16 * (m + 1)]


_BR = 1024


@functools.partial(jax.jit, static_argnames=("interpret",))
def kernel(x, p, interpret=False):
    batch, n_len = x.shape

    t16 = pl.pallas_call(
        _prep_kernel,
        out_shape=jax.ShapeDtypeStruct((96, _N_LENGTH), jnp.float32),
        interpret=interpret,
    )(p, jnp.asarray(_ELMT), jnp.asarray(_TRI))
    t128 = jnp.concatenate([jnp.tile(t16, (1, 8)), jnp.asarray(_D128)], axis=0)

    grid = batch // (_BR * 8)
    y = pl.pallas_call(
        _main_kernel,
        grid=(grid,),
        in_specs=[
            pl.BlockSpec((128, 128), lambda i: (0, 0)),
            pl.BlockSpec((_BR * 8, n_len), lambda i: (i, 0)),
        ],
        out_specs=pl.BlockSpec((_BR * 8, n_len), lambda i: (i, 0)),
        out_shape=jax.ShapeDtypeStruct((batch, n_len), jnp.float32),
        compiler_params=pltpu.CompilerParams(
            dimension_semantics=("parallel",),
        ),
        interpret=interpret,
    )(t128, x)
    return y


# speedup vs baseline: 2849.6920x; 5.1886x over previous
import functools

import jax
import jax.numpy as jnp
import numpy as np
from jax.experimental import pallas as pl
from jax.experimental.pallas import tpu as pltpu

_N_BINS = 32
_R = 1.2
_BOUND = 50.0
_N_LENGTH = 16


def _np_mesh_constants():
    m = _N_BINS / 2
    x1L = _BOUND * (_R - 1.0) / (_R**m - 1.0)
    index = np.arange(0, _N_BINS + 1, dtype=np.float64).reshape(-1, 1) - m
    xr = np.where(index >= 0,
                  (1.0 - _R**index) / (1.0 - _R),
                  (1.0 - _R**np.abs(index)) / (1.0 - _R))
    xr = np.where(index >= 0, x1L * xr, -x1L * xr)
    xr = (xr + _BOUND) / 2.0 / _BOUND
    x1L_s = x1L / 2.0 / _BOUND
    mesh = np.concatenate([np.zeros((1, 1)), xr[1:-1, 0:1], np.ones((1, 1))], 0)
    elmt = (mesh[1:] - mesh[:-1]).reshape(-1, 1)
    return (mesh.astype(np.float32), elmt.astype(np.float32),
            np.float32(x1L_s))


_MESH, _ELMT, _X1L = _np_mesh_constants()
_ELMT_ROW = _ELMT.reshape(1, _N_BINS)
_W_ROW = ((_ELMT[:-1, 0] + _ELMT[1:, 0]) / 2.0).reshape(1, _N_BINS - 1)
_MESH_ROW = _MESH[:_N_BINS, 0].reshape(1, _N_BINS)
_TRIU = (np.arange(_N_BINS)[:, None] < np.arange(_N_BINS)[None, :]).astype(
    np.float32)
_ACOEF = float((_R - 1.0) / _X1L)
_INV_LOG_R = float(1.0 / np.log(_R))


def _prep_kernel(p_ref, elmt_ref, w_ref, mesh_ref, triu_ref, t_ref):
    pt = p_ref[...]
    ep = jnp.exp(pt)
    s = jnp.sum(ep * w_ref[...], axis=1, keepdims=True)
    px = ((1.0 - float(_ELMT[0, 0])) / s) * ep
    one = jnp.ones((_N_LENGTH, 1), jnp.float32)
    pdf = jnp.concatenate([one, px, one], axis=1)
    elmt = elmt_ref[...]
    cell = (pdf[:, :-1] + pdf[:, 1:]) / 2.0 * elmt
    f_ref = jnp.dot(cell, triu_ref[...],
                    preferred_element_type=jnp.float32,
                    precision=jax.lax.Precision.HIGHEST)
    b = pdf[:, :_N_BINS]
    c = (pdf[:, 1:] - pdf[:, :-1]) / (2.0 * elmt)
    d = jnp.broadcast_to(mesh_ref[...], (_N_LENGTH, _N_BINS))
    t_ref[...] = jnp.concatenate([f_ref, b, c, d], axis=1)


def _main_kernel(t_ref, x_ref, o_ref):
    t = t_ref[...]
    x = x_ref[...]
    xs = (x + _BOUND) * (0.5 / _BOUND)
    d = xs - 0.5
    tt = jnp.abs(d) * _ACOEF + 1.0
    mvar = jnp.floor(jnp.log(tt) * _INV_LOG_R)
    kf = jnp.where(d >= 0, 16.0 + mvar, 15.0 - mvar)
    inr = (kf >= 0.0) & (kf <= 31.0)
    ki = jnp.clip(kf, 0.0, 31.0).astype(jnp.int32)

    def gather(off):
        return jnp.take_along_axis(t, ki + off, axis=1,
                                   mode="promise_in_bounds")

    a = gather(0)
    b = gather(32)
    c = gather(64)
    dm = gather(96)
    xm = xs - dm
    y = a + xm * (b + xm * c)
    res = jnp.where(inr, y, xs)
    o_ref[...] = res * (2.0 * _BOUND) - _BOUND


_W = 32768


@functools.partial(jax.jit, static_argnames=("interpret",))
def kernel(x, p, interpret=False):
    batch, n_len = x.shape
    xt = x.T
    pt = p.T

    t = pl.pallas_call(
        _prep_kernel,
        out_shape=jax.ShapeDtypeStruct((_N_LENGTH, 128), jnp.float32),
        interpret=interpret,
    )(pt, jnp.asarray(_ELMT_ROW), jnp.asarray(_W_ROW),
      jnp.asarray(_MESH_ROW), jnp.asarray(_TRIU))

    grid = batch // _W
    yt = pl.pallas_call(
        _main_kernel,
        grid=(grid,),
        in_specs=[
            pl.BlockSpec((_N_LENGTH, 128), lambda i: (0, 0)),
            pl.BlockSpec((_N_LENGTH, _W), lambda i: (0, i)),
        ],
        out_specs=pl.BlockSpec((_N_LENGTH, _W), lambda i: (0, i)),
        out_shape=jax.ShapeDtypeStruct((n_len, batch), jnp.float32),
        compiler_params=pltpu.CompilerParams(
            dimension_semantics=("parallel",),
        ),
        interpret=interpret,
    )(t, xt)
    return yt.T


# speedup vs baseline: 4862.3168x; 1.7063x over previous
import functools

import jax
import jax.numpy as jnp
import numpy as np
from jax.experimental import pallas as pl
from jax.experimental.pallas import tpu as pltpu

_N_BINS = 32
_R = 1.2
_BOUND = 50.0
_N_LENGTH = 16


def _np_mesh_constants():
    m = _N_BINS / 2
    x1L = _BOUND * (_R - 1.0) / (_R**m - 1.0)
    index = np.arange(0, _N_BINS + 1, dtype=np.float64).reshape(-1, 1) - m
    xr = np.where(index >= 0,
                  (1.0 - _R**index) / (1.0 - _R),
                  (1.0 - _R**np.abs(index)) / (1.0 - _R))
    xr = np.where(index >= 0, x1L * xr, -x1L * xr)
    xr = (xr + _BOUND) / 2.0 / _BOUND
    x1L_s = x1L / 2.0 / _BOUND
    mesh = np.concatenate([np.zeros((1, 1)), xr[1:-1, 0:1], np.ones((1, 1))], 0)
    elmt = (mesh[1:] - mesh[:-1]).reshape(-1, 1)
    return (mesh.astype(np.float32), elmt.astype(np.float32),
            np.float32(x1L_s))


_MESH, _ELMT, _X1L = _np_mesh_constants()
_ELMT_ROW = _ELMT.reshape(1, _N_BINS)
_W_ROW = ((_ELMT[:-1, 0] + _ELMT[1:, 0]) / 2.0).reshape(1, _N_BINS - 1)
_MESH_ROW = _MESH[:_N_BINS, 0].reshape(1, _N_BINS)
_TRIU = (np.arange(_N_BINS)[:, None] < np.arange(_N_BINS)[None, :]).astype(
    np.float32)
_ACOEF = float((_R - 1.0) / _X1L)
_INV_LOG_R = float(1.0 / np.log(_R))


def _prep_kernel(p_ref, elmt_ref, w_ref, mesh_ref, triu_ref, t_ref):
    pt = p_ref[...]
    ep = jnp.exp(pt)
    s = jnp.sum(ep * w_ref[...], axis=1, keepdims=True)
    px = ((1.0 - float(_ELMT[0, 0])) / s) * ep
    one = jnp.ones((_N_LENGTH, 1), jnp.float32)
    pdf = jnp.concatenate([one, px, one], axis=1)
    elmt = elmt_ref[...]
    cell = (pdf[:, :-1] + pdf[:, 1:]) / 2.0 * elmt
    f_ref = jnp.dot(cell, triu_ref[...],
                    preferred_element_type=jnp.float32,
                    precision=jax.lax.Precision.HIGHEST)
    a = f_ref * (2.0 * _BOUND) - _BOUND
    b = pdf[:, :_N_BINS] * (2.0 * _BOUND)
    c = (pdf[:, 1:] - pdf[:, :-1]) / (2.0 * elmt) * (2.0 * _BOUND)
    ub = jax.lax.bitcast_convert_type(b, jnp.uint32)
    uc = jax.lax.bitcast_convert_type(c, jnp.uint32)
    ub = (ub + 0x8000) & jnp.uint32(0xFFFF0000)
    uc = (uc + 0x8000) >> 16
    bc = jax.lax.bitcast_convert_type(ub | uc, jnp.float32)
    t_ref[...] = jnp.concatenate([a, bc], axis=1)


_LOG2R = float(np.log2(_R))
_INV_A = float(1.0 / _ACOEF)


def _main_kernel(t_ref, x_ref, o_ref):
    t = t_ref[...]
    x = x_ref[...]
    d = x * (0.5 / _BOUND)
    ad = jnp.abs(d)
    tt = ad * _ACOEF + 1.0
    pos = d >= 0
    mf = jnp.trunc(jnp.log(tt) * _INV_LOG_R)
    km = mf.astype(jnp.int32)
    ki = 16 + jnp.where(pos, km, ~km)
    inr = (ki & ~31) == 0
    kg = ki & 31

    def gather(off):
        return jnp.take_along_axis(t, kg + off, axis=1,
                                   mode="promise_in_bounds")

    a = gather(0)
    gbc = jax.lax.bitcast_convert_type(gather(32), jnp.uint32)
    b = jax.lax.bitcast_convert_type(gbc & jnp.uint32(0xFFFF0000),
                                     jnp.float32)
    c = jax.lax.bitcast_convert_type(gbc << 16, jnp.float32)
    mprime = jnp.where(pos, mf, mf + 1.0)
    g = (jnp.exp2(mprime * _LOG2R) - 1.0) * _INV_A
    axm = ad - g
    xm = jnp.where(pos, axm, -axm)
    y = a + xm * (b + xm * c)
    o_ref[...] = jnp.where(inr, y, x)


_W = 32768


@functools.partial(jax.jit, static_argnames=("interpret",))
def kernel(x, p, interpret=False):
    batch, n_len = x.shape
    xt = x.T
    pt = p.T

    t = pl.pallas_call(
        _prep_kernel,
        out_shape=jax.ShapeDtypeStruct((_N_LENGTH, 64), jnp.float32),
        interpret=interpret,
    )(pt, jnp.asarray(_ELMT_ROW), jnp.asarray(_W_ROW),
      jnp.asarray(_MESH_ROW), jnp.asarray(_TRIU))

    grid = batch // _W
    yt = pl.pallas_call(
        _main_kernel,
        grid=(grid,),
        in_specs=[
            pl.BlockSpec((_N_LENGTH, 64), lambda i: (0, 0)),
            pl.BlockSpec((_N_LENGTH, _W), lambda i: (0, i)),
        ],
        out_specs=pl.BlockSpec((_N_LENGTH, _W), lambda i: (0, i)),
        out_shape=jax.ShapeDtypeStruct((n_len, batch), jnp.float32),
        compiler_params=pltpu.CompilerParams(
            dimension_semantics=("parallel",),
        ),
        interpret=interpret,
    )(t, xt)
    return yt.T
